# baseline (speedup 1.0000x reference)
import functools

import jax
import jax.numpy as jnp
from jax import lax
from jax.experimental import pallas as pl
from jax.experimental.pallas import tpu as pltpu
from jax.experimental.pallas import tpu_sc as plsc

DIM = 64
BATCH = 16384
ROWS = 1000000
FLAT = DIM * ROWS

_info = plsc.get_sparse_core_info()
_NC = _info.num_cores
_NS = _info.num_subcores
_NW = _NC * _NS
_BPW = BATCH // _NW
_CH = 128
_NCH = _BPW // _CH

_mesh = plsc.VectorSubcoreMesh(core_axis_name="c", subcore_axis_name="s")


@functools.partial(
    pl.kernel,
    mesh=_mesh,
    out_type=(
        jax.ShapeDtypeStruct((DIM, BATCH), jnp.float32),
        jax.ShapeDtypeStruct((DIM, BATCH), jnp.float32),
    ),
    scratch_types=[
        pltpu.VMEM((_BPW,), jnp.int32),
        pltpu.VMEM((DIM, _BPW), jnp.float32),
        pltpu.VMEM((DIM, _BPW), jnp.float32),
        pltpu.SemaphoreType.DMA,
        pltpu.SemaphoreType.DMA,
        pltpu.SemaphoreType.DMA,
    ],
)
def _dual_gather(edge_hbm, node_hbm, idx_hbm, out_e, out_n,
                 idx_v, ebuf, nbuf, sem_e, sem_n, sem_w):
    wid = lax.axis_index("s") * _NC + lax.axis_index("c")
    base = wid * _BPW
    pltpu.async_copy(idx_hbm.at[pl.ds(base, _BPW)], idx_v, sem_w).wait()

    def issue_plane(j):
        src_e = edge_hbm.at[pl.ds(j * ROWS, ROWS)]
        src_n = node_hbm.at[pl.ds(j * ROWS, ROWS)]
        for ch in range(_NCH):
            ids = idx_v.at[pl.ds(ch * _CH, _CH)]
            dst = pl.ds(ch * _CH, _CH)
            pltpu.async_copy(src_e.at[ids], ebuf.at[j, dst], sem_e)
            pltpu.async_copy(src_n.at[ids], nbuf.at[j, dst], sem_n)

    def drain_plane(j):
        src_e = edge_hbm.at[pl.ds(0, ROWS)]
        src_n = node_hbm.at[pl.ds(0, ROWS)]
        for ch in range(_NCH):
            ids = idx_v.at[pl.ds(ch * _CH, _CH)]
            dst = pl.ds(ch * _CH, _CH)
            pltpu.make_async_copy(src_e.at[ids], ebuf.at[j, dst], sem_e).wait()
            pltpu.make_async_copy(src_n.at[ids], nbuf.at[j, dst], sem_n).wait()

    def body(j, carry):
        @pl.when(j < DIM)
        def _():
            issue_plane(j)
        @pl.when(j >= 2)
        def _():
            drain_plane(j - 2)
        return carry

    lax.fori_loop(0, DIM + 2, body, 0)

    cols = pl.ds(base, _BPW)
    pltpu.async_copy(ebuf, out_e.at[:, cols], sem_w)
    pltpu.async_copy(nbuf, out_n.at[:, cols], sem_w)
    pltpu.make_async_copy(ebuf, out_e.at[:, cols], sem_w).wait()
    pltpu.make_async_copy(nbuf, out_n.at[:, cols], sem_w).wait()


def kernel(edge_emb, node_emb, feat):
    ef = edge_emb.T.reshape(FLAT)
    nf = node_emb.T.reshape(FLAT)
    oe_t, on_t = _dual_gather(ef, nf, feat)
    return oe_t.T, on_t.T


# speedup vs baseline: 8.4722x; 8.4722x over previous
import functools

import jax
import jax.numpy as jnp
from jax import lax
from jax.experimental import pallas as pl
from jax.experimental.pallas import tpu as pltpu
from jax.experimental.pallas import tpu_sc as plsc

DIM = 64
BATCH = 16384
ROWS = 1000000
FLAT = DIM * ROWS
_SLICE = FLAT - 56

_info = plsc.get_sparse_core_info()
_NC = _info.num_cores
_NS = _info.num_subcores
_NW = _NC * _NS
_BPW = BATCH // _NW
_CH = 128
_NCH = _BPW // _CH

_mesh = plsc.VectorSubcoreMesh(core_axis_name="c", subcore_axis_name="s")


@functools.partial(
    pl.kernel,
    mesh=_mesh,
    out_type=(
        jax.ShapeDtypeStruct((DIM, BATCH), jnp.float32),
        jax.ShapeDtypeStruct((DIM, BATCH), jnp.float32),
    ),
    scratch_types=[
        pltpu.VMEM((_BPW,), jnp.int32),
        pltpu.VMEM((8, _BPW), jnp.int32),
        pltpu.VMEM((DIM, _BPW), jnp.float32),
        pltpu.VMEM((DIM, _BPW), jnp.float32),
        pltpu.SemaphoreType.DMA,
        pltpu.SemaphoreType.DMA,
        pltpu.SemaphoreType.DMA,
    ],
)
def _dual_gather(edge_hbm, node_hbm, idx_hbm, out_e, out_n,
                 idx_v, sbuf, ebuf, nbuf, sem_e, sem_n, sem_w):
    wid = lax.axis_index("s") * _NC + lax.axis_index("c")
    base = wid * _BPW
    pltpu.async_copy(idx_hbm.at[pl.ds(base, _BPW)], idx_v, sem_w).wait()

    def scale(v, carry):
        vec = idx_v[pl.ds(v * 16, 16)] * DIM
        for jr in range(8):
            sbuf[jr, pl.ds(v * 16, 16)] = vec + jr
        return carry

    lax.fori_loop(0, _BPW // 16, scale, 0)

    def issue_plane(j8, jr):
        src_e = edge_hbm.at[pl.ds(j8 * 8, _SLICE)]
        src_n = node_hbm.at[pl.ds(j8 * 8, _SLICE)]
        j = j8 * 8 + jr
        for ch in range(_NCH):
            ids = sbuf.at[jr, pl.ds(ch * _CH, _CH)]
            dst = pl.ds(ch * _CH, _CH)
            pltpu.async_copy(src_e.at[ids], ebuf.at[j, dst], sem_e)
            pltpu.async_copy(src_n.at[ids], nbuf.at[j, dst], sem_n)

    def drain_plane(j8, jr):
        src_e = edge_hbm.at[pl.ds(j8 * 8, _SLICE)]
        src_n = node_hbm.at[pl.ds(j8 * 8, _SLICE)]
        j = j8 * 8 + jr
        for ch in range(_NCH):
            ids = sbuf.at[jr, pl.ds(ch * _CH, _CH)]
            dst = pl.ds(ch * _CH, _CH)
            pltpu.make_async_copy(src_e.at[ids], ebuf.at[j, dst], sem_e).wait()
            pltpu.make_async_copy(src_n.at[ids], nbuf.at[j, dst], sem_n).wait()

    def body(j, carry):
        @pl.when(j < DIM)
        def _():
            issue_plane(lax.div(j, 8), lax.rem(j, 8))
        @pl.when(j >= 2)
        def _():
            jp = j - 2
            drain_plane(lax.div(jp, 8), lax.rem(jp, 8))
        return carry

    lax.fori_loop(0, DIM + 2, body, 0)

    cols = pl.ds(base, _BPW)
    pltpu.async_copy(ebuf, out_e.at[:, cols], sem_w)
    pltpu.async_copy(nbuf, out_n.at[:, cols], sem_w)
    pltpu.make_async_copy(ebuf, out_e.at[:, cols], sem_w).wait()
    pltpu.make_async_copy(nbuf, out_n.at[:, cols], sem_w).wait()


def kernel(edge_emb, node_emb, feat):
    ef = edge_emb.reshape(FLAT)
    nf = node_emb.reshape(FLAT)
    oe_t, on_t = _dual_gather(ef, nf, feat)
    return oe_t.T, on_t.T


# speedup vs baseline: 8.4863x; 1.0017x over previous
import functools

import jax
import jax.numpy as jnp
from jax import lax
from jax.experimental import pallas as pl
from jax.experimental.pallas import tpu as pltpu
from jax.experimental.pallas import tpu_sc as plsc

DIM = 64
BATCH = 16384
ROWS = 1000000
FLAT = DIM * ROWS
_SLICE = FLAT - 56

_info = plsc.get_sparse_core_info()
_NC = _info.num_cores
_NS = _info.num_subcores
_NW = _NC * _NS
_BPW = BATCH // _NW
_CH = 128
_NCH = _BPW // _CH

_mesh = plsc.VectorSubcoreMesh(core_axis_name="c", subcore_axis_name="s")


@functools.partial(
    pl.kernel,
    mesh=_mesh,
    out_type=(
        jax.ShapeDtypeStruct((DIM, BATCH), jnp.float32),
        jax.ShapeDtypeStruct((DIM, BATCH), jnp.float32),
    ),
    scratch_types=[
        pltpu.VMEM((_BPW,), jnp.int32),
        pltpu.VMEM((8, _BPW), jnp.int32),
        pltpu.VMEM((DIM, _BPW), jnp.float32),
        pltpu.VMEM((DIM, _BPW), jnp.float32),
        pltpu.SemaphoreType.DMA,
        pltpu.SemaphoreType.DMA,
        pltpu.SemaphoreType.DMA,
    ],
    compiler_params=pltpu.CompilerParams(skip_device_barrier=True),
)
def _dual_gather(edge_hbm, node_hbm, idx_hbm, out_e, out_n,
                 idx_v, sbuf, ebuf, nbuf, sem_e, sem_n, sem_w):
    wid = lax.axis_index("s") * _NC + lax.axis_index("c")
    base = wid * _BPW
    pltpu.async_copy(idx_hbm.at[pl.ds(base, _BPW)], idx_v, sem_w).wait()

    def scale(v, carry):
        vec = idx_v[pl.ds(v * 16, 16)] * DIM
        for jr in range(8):
            sbuf[jr, pl.ds(v * 16, 16)] = vec + jr
        return carry

    lax.fori_loop(0, _BPW // 16, scale, 0)

    def issue_plane(j8, jr):
        src_e = edge_hbm.at[pl.ds(j8 * 8, _SLICE)]
        src_n = node_hbm.at[pl.ds(j8 * 8, _SLICE)]
        j = j8 * 8 + jr
        for ch in range(_NCH):
            ids = sbuf.at[jr, pl.ds(ch * _CH, _CH)]
            dst = pl.ds(ch * _CH, _CH)
            pltpu.async_copy(src_e.at[ids], ebuf.at[j, dst], sem_e)
            pltpu.async_copy(src_n.at[ids], nbuf.at[j, dst], sem_n)

    def drain_plane(j8, jr):
        src_e = edge_hbm.at[pl.ds(j8 * 8, _SLICE)]
        src_n = node_hbm.at[pl.ds(j8 * 8, _SLICE)]
        j = j8 * 8 + jr
        for ch in range(_NCH):
            ids = sbuf.at[jr, pl.ds(ch * _CH, _CH)]
            dst = pl.ds(ch * _CH, _CH)
            pltpu.make_async_copy(src_e.at[ids], ebuf.at[j, dst], sem_e).wait()
            pltpu.make_async_copy(src_n.at[ids], nbuf.at[j, dst], sem_n).wait()

    def body(j, carry):
        @pl.when(j < DIM)
        def _():
            issue_plane(lax.div(j, 8), lax.rem(j, 8))
        @pl.when(j >= 2)
        def _():
            jp = j - 2
            drain_plane(lax.div(jp, 8), lax.rem(jp, 8))
        return carry

    lax.fori_loop(0, DIM + 2, body, 0)

    cols = pl.ds(base, _BPW)
    pltpu.async_copy(ebuf, out_e.at[:, cols], sem_w)
    pltpu.async_copy(nbuf, out_n.at[:, cols], sem_w)
    pltpu.make_async_copy(ebuf, out_e.at[:, cols], sem_w).wait()
    pltpu.make_async_copy(nbuf, out_n.at[:, cols], sem_w).wait()


def kernel(edge_emb, node_emb, feat):
    ef = edge_emb.reshape(FLAT)
    nf = node_emb.reshape(FLAT)
    oe_t, on_t = _dual_gather(ef, nf, feat)
    return oe_t.T, on_t.T


# speedup vs baseline: 8.9232x; 1.0515x over previous
import functools

import jax
import jax.numpy as jnp
from jax import lax
from jax.experimental import pallas as pl
from jax.experimental.pallas import tpu as pltpu
from jax.experimental.pallas import tpu_sc as plsc

DIM = 64
BATCH = 16384

_info = plsc.get_sparse_core_info()
_NC = _info.num_cores
_NS = _info.num_subcores
_NW = _NC * _NS
_BPW = BATCH // _NW
_CH = 128
_NCH = _BPW // _CH

_mesh = plsc.VectorSubcoreMesh(core_axis_name="c", subcore_axis_name="s")


@functools.partial(
    pl.kernel,
    mesh=_mesh,
    out_type=(
        jax.ShapeDtypeStruct((BATCH, DIM), jnp.float32),
        jax.ShapeDtypeStruct((BATCH, DIM), jnp.float32),
    ),
    scratch_types=[
        pltpu.VMEM((_BPW,), jnp.int32),
        pltpu.VMEM((_BPW, DIM), jnp.float32),
        pltpu.VMEM((_BPW, DIM), jnp.float32),
        pltpu.SemaphoreType.DMA,
        pltpu.SemaphoreType.DMA,
    ],
    compiler_params=pltpu.CompilerParams(use_tc_tiling_on_sc=False),
)
def _dual_gather(edge_hbm, node_hbm, feat_hbm, out_e, out_n,
                 idx_v, erows, nrows, sem_e, sem_n):
    wid = lax.axis_index("s") * _NC + lax.axis_index("c")
    base = wid * _BPW
    pltpu.sync_copy(feat_hbm.at[pl.ds(base, _BPW)], idx_v)
    copies = []
    for j in range(_NCH):
        sl = pl.ds(j * _CH, _CH)
        ce = pltpu.async_copy(edge_hbm.at[idx_v.at[sl]], erows.at[sl], sem_e)
        cn = pltpu.async_copy(node_hbm.at[idx_v.at[sl]], nrows.at[sl], sem_n)
        copies.append((ce, cn))
    for j, (ce, cn) in enumerate(copies):
        sl = pl.ds(j * _CH, _CH)
        out_sl = pl.ds(base + j * _CH, _CH)
        ce.wait()
        pltpu.sync_copy(erows.at[sl], out_e.at[out_sl])
        cn.wait()
        pltpu.sync_copy(nrows.at[sl], out_n.at[out_sl])


def kernel(edge_emb, node_emb, feat):
    return _dual_gather(edge_emb, node_emb, feat)
